# Initial kernel scaffold; baseline (speedup 1.0000x reference)
import jax
import jax.numpy as jnp
from jax.experimental import pallas as pl


def kernel(user_id, pos_items, neg_items, user_table, item_table):
    raise NotImplementedError("write your pallas kernel here")


# speedup vs baseline: 24.8476x; 24.8476x over previous
import functools

import jax
import jax.numpy as jnp
from jax import lax
from jax.experimental import pallas as pl
from jax.experimental.pallas import tpu as pltpu
from jax.experimental.pallas import tpu_sc as plsc

B = 16384
N_NEG = 200
D = 64
NC = 2
NS = 16
NW = NC * NS
BPW = B // NW
CB = 8
NCHUNK = BPW // CB
GW = 100
GPC = CB * N_NEG // GW
NPAD = N_NEG + 8


def _dot4(row_ref, r, q0, q1, q2, q3):
    a = row_ref[r, pl.ds(0, 16)] * q0
    a = a + row_ref[r, pl.ds(16, 16)] * q1
    a = a + row_ref[r, pl.ds(32, 16)] * q2
    a = a + row_ref[r, pl.ds(48, 16)] * q3
    return jnp.sum(a)


def _sc_body(uid_hbm, pos_hbm, negf_hbm, utab_hbm, itab_hbm,
             pos_out, neg_out,
             nidx_v, qidx_v, pidx_v, qrows_v, prows_v, rows_v,
             nscore_v, pscore_v, sem):
    cid = lax.axis_index("c")
    sid = lax.axis_index("s")
    wid = sid * NC + cid
    base = pl.multiple_of(wid * BPW, BPW)
    iota = lax.iota(jnp.int32, 16)
    mask8 = iota < 8

    def chunk_body(c, _):
        cb = pl.multiple_of(base + c * CB, CB)
        pltpu.sync_copy(uid_hbm.at[pl.ds(cb, CB)], qidx_v)
        pltpu.sync_copy(pos_hbm.at[pl.ds(cb, CB)], pidx_v)
        pltpu.sync_copy(negf_hbm.at[pl.ds(cb * (N_NEG // GW), GPC)], nidx_v)
        cps = [pltpu.async_copy(utab_hbm.at[qidx_v], qrows_v, sem),
               pltpu.async_copy(itab_hbm.at[pidx_v], prows_v, sem)]
        for g in range(GPC):
            cps.append(pltpu.async_copy(itab_hbm.at[nidx_v.at[g]],
                                        rows_v.at[pl.ds(g * GW, GW)], sem))
        for cp in cps:
            cp.wait()
        pos_vec = jnp.zeros((16,), jnp.float32)
        for b in range(CB):
            q0 = qrows_v[b, pl.ds(0, 16)]
            q1 = qrows_v[b, pl.ds(16, 16)]
            q2 = qrows_v[b, pl.ds(32, 16)]
            q3 = qrows_v[b, pl.ds(48, 16)]
            ps = _dot4(prows_v, b, q0, q1, q2, q3)
            pos_vec = jnp.where(iota == b, ps, pos_vec)

            def nbody(g, _, b=b, q0=q0, q1=q1, q2=q2, q3=q3):
                ss = [_dot4(rows_v, b * N_NEG + g * 8 + u, q0, q1, q2, q3)
                      for u in range(8)]
                vec = jnp.full((16,), ss[0], jnp.float32)
                for u in range(1, 8):
                    vec = jnp.where(iota == u, ss[u], vec)
                plsc.store_compressed(
                    nscore_v.at[pl.ds(b * NPAD + g * 8, 16)],
                    vec, mask=mask8)
                return 0

            lax.fori_loop(0, N_NEG // 8, nbody, 0)
        plsc.store_compressed(pscore_v.at[pl.ds(c * CB, 16)], pos_vec,
                              mask=mask8)
        for b in range(CB):
            pltpu.sync_copy(nscore_v.at[pl.ds(b * NPAD, N_NEG)],
                            neg_out.at[cb + b])
        return 0

    lax.fori_loop(0, NCHUNK, chunk_body, 0)
    pltpu.sync_copy(pscore_v.at[pl.ds(0, BPW)], pos_out.at[pl.ds(base, BPW)])


@jax.jit
def _mf_scores(user_id, pos_items, neg_flat, user_table, item_table):
    mesh = plsc.VectorSubcoreMesh(core_axis_name="c", subcore_axis_name="s")
    f = functools.partial(
        pl.kernel,
        mesh=mesh,
        compiler_params=pltpu.CompilerParams(needs_layout_passes=False,
                                             use_tc_tiling_on_sc=False),
        out_type=[jax.ShapeDtypeStruct((B,), jnp.float32),
                  jax.ShapeDtypeStruct((B, N_NEG), jnp.float32)],
        scratch_types=[
            pltpu.VMEM((GPC, GW), jnp.int32),
            pltpu.VMEM((CB,), jnp.int32),
            pltpu.VMEM((CB,), jnp.int32),
            pltpu.VMEM((CB, D), jnp.float32),
            pltpu.VMEM((CB, D), jnp.float32),
            pltpu.VMEM((CB * N_NEG, D), jnp.float32),
            pltpu.VMEM((CB * NPAD,), jnp.float32),
            pltpu.VMEM((BPW + 8,), jnp.float32),
            pltpu.SemaphoreType.DMA,
        ],
    )(_sc_body)
    return f(user_id, pos_items, neg_flat, user_table, item_table)


def kernel(user_id, pos_items, neg_items, user_table, item_table):
    user_id = user_id.astype(jnp.int32)
    pos_items = pos_items.astype(jnp.int32)
    neg_flat = neg_items.astype(jnp.int32).reshape(B * N_NEG // GW, GW)
    pos_score, neg_score = _mf_scores(user_id, pos_items, neg_flat,
                                      user_table, item_table)
    return pos_score, neg_score
